# baseline (speedup 1.0000x reference)
import functools

import jax
import jax.numpy as jnp
from jax import lax
from jax.experimental import pallas as pl
from jax.experimental.pallas import tpu as pltpu
from jax.experimental.pallas import tpu_sc as plsc

N = 10000
E = 320000
D = 128
NC = 2
NS = 16
NW = NC * NS
CHUNK = 128
CPS = 8
SUP = CPS * CHUNK
PT = ((E + NW - 1) // NW + SUP - 1) // SUP * SUP
EPAD = PT * NW
NCHUNK = PT // CHUNK
NSUP = PT // SUP
NPAD = 10240
STRIPE = NPAD // NS

_mesh = plsc.VectorSubcoreMesh(core_axis_name="c", subcore_axis_name="s",
                               num_cores=NC, num_subcores=NS)


def _zero_fill(buf, nrows, ncols):
    z16 = jnp.zeros((16,), jnp.float32)

    def fill(i, _):
        buf[i // (ncols // 16), pl.ds((i % (ncols // 16)) * 16, 16)] = z16
        return 0

    lax.fori_loop(0, nrows * (ncols // 16), fill, 0)


def _seg_core(src_hbm, dst_hbm, feat_hbm, out_sum, sum_sh,
              sidx, didx, rows0, rows1, gsem0, gsem1):
    c = lax.axis_index("c")
    s = lax.axis_index("s")

    _zero_fill(rows0, CHUNK, D)
    row0 = s * STRIPE

    def zcopy(j, _):
        pltpu.sync_copy(rows0, sum_sh.at[pl.ds(row0 + j * CHUNK, CHUNK), :])
        return 0

    lax.fori_loop(0, STRIPE // CHUNK, zcopy, 0)
    plsc.subcore_barrier()

    g = c * NS + s
    cbase = g * NCHUNK
    pltpu.sync_copy(src_hbm.at[pl.ds(cbase, CPS), :], sidx)
    pltpu.sync_copy(dst_hbm.at[pl.ds(cbase, CPS), :], didx)
    bufs = (rows0, rows1)
    sems = (gsem0, gsem1)

    def qbody(q, _):
        def fire(k):
            return pltpu.async_copy(feat_hbm.at[sidx.at[k]],
                                    bufs[k % 2], sems[k % 2])

        descs = [fire(0)] + [None] * (CPS - 1)
        for k in range(CPS):
            if k + 1 < CPS:
                descs[k + 1] = fire(k + 1)
            descs[k].wait()
            pltpu.sync_copy(bufs[k % 2], sum_sh.at[didx.at[k]], add=True)

        @pl.when(q + 1 < NSUP)
        def _():
            nxt = cbase + (q + 1) * CPS
            pltpu.sync_copy(src_hbm.at[pl.ds(nxt, CPS), :], sidx)
            pltpu.sync_copy(dst_hbm.at[pl.ds(nxt, CPS), :], didx)

        return 0

    lax.fori_loop(0, NSUP, qbody, 0)
    plsc.subcore_barrier()

    pltpu.sync_copy(sum_sh.at[pl.ds(row0, STRIPE), :],
                    out_sum.at[c, pl.ds(row0, STRIPE), :])


@functools.partial(
    pl.kernel,
    out_type=jax.ShapeDtypeStruct((NW, NPAD), jnp.float32),
    mesh=_mesh,
    scratch_types=dict(
        didx=pltpu.VMEM((SUP,), jnp.int32),
        cntv=pltpu.VMEM((NPAD,), jnp.float32),
    ),
    compiler_params=pltpu.CompilerParams(needs_layout_passes=False),
)
def _degree(dst_hbm, out_cnt, didx, cntv):
    c = lax.axis_index("c")
    s = lax.axis_index("s")
    z16 = jnp.zeros((16,), jnp.float32)

    def czf(i, _):
        cntv[pl.ds(i * 16, 16)] = z16
        return 0

    lax.fori_loop(0, NPAD // 16, czf, 0)
    g = c * NS + s
    ebase = g * PT
    one16 = jnp.ones((16,), jnp.float32)

    def esup(i, _):
        pltpu.sync_copy(dst_hbm.at[pl.ds(ebase + i * SUP, SUP)], didx)

        def cadd(j, _):
            v = didx[pl.ds(j * 16, 16)]
            plsc.addupdate_scatter(cntv, [v], one16)
            return 0

        lax.fori_loop(0, SUP // 16, cadd, 0)
        return 0

    lax.fori_loop(0, NSUP, esup, 0)
    pltpu.sync_copy(cntv, out_cnt.at[g])


@functools.partial(
    pl.kernel,
    out_type=jax.ShapeDtypeStruct((NC, NPAD, D), jnp.float32),
    mesh=_mesh,
    scratch_types=dict(
        sidx=pltpu.VMEM((CPS, CHUNK), jnp.int32),
        didx=pltpu.VMEM((CPS, CHUNK), jnp.int32),
        rows0=pltpu.VMEM((CHUNK, D), jnp.float32),
        rows1=pltpu.VMEM((CHUNK, D), jnp.float32),
        sum_sh=pltpu.VMEM_SHARED((NPAD, D), jnp.float32),
        gsem0=pltpu.SemaphoreType.DMA,
        gsem1=pltpu.SemaphoreType.DMA,
    ),
)
def _seg_sum(src_hbm, dst_hbm, feat_hbm, out_sum,
             sidx, didx, rows0, rows1, sum_sh, gsem0, gsem1):
    _seg_core(src_hbm, dst_hbm, feat_hbm, out_sum, sum_sh,
              sidx, didx, rows0, rows1, gsem0, gsem1)


BN = 1024


def _layer1_body(p0, p1, cn, x, wl, wr, b, out):
    cnt = jnp.maximum(jnp.sum(cn[...], axis=0), 1.0)
    mean = (p0[...] + p1[...]) / cnt[:, None]
    acc = jnp.dot(mean, wl[...], preferred_element_type=jnp.float32)
    acc = acc + jnp.dot(x[...], wr[...], preferred_element_type=jnp.float32)
    out[...] = jnp.maximum(acc + b[...], 0.0)


def _layer2_body(q0, q1, cn, h, wl, wr, b, wh, bh, out):
    cnt = jnp.maximum(jnp.sum(cn[...], axis=0), 1.0)
    mean = (q0[...] + q1[...]) / cnt[:, None]
    acc = jnp.dot(mean, wl[...], preferred_element_type=jnp.float32)
    acc = acc + jnp.dot(h[...], wr[...], preferred_element_type=jnp.float32)
    h2 = jnp.maximum(acc + b[...], 0.0)
    out[...] = jnp.dot(h2, wh[...], preferred_element_type=jnp.float32) + bh[...]


def _row_spec(w):
    return pl.BlockSpec((BN, w), lambda i: (i, 0))


def _cnt_spec():
    return pl.BlockSpec((NW, BN), lambda i: (0, i))


def _full_spec(r, cdim):
    return pl.BlockSpec((r, cdim), lambda i: (0, 0))


_layer1 = pl.pallas_call(
    _layer1_body,
    grid=(NPAD // BN,),
    in_specs=[_row_spec(D), _row_spec(D), _cnt_spec(),
              _row_spec(D), _full_spec(D, D), _full_spec(D, D), _full_spec(1, D)],
    out_specs=_row_spec(D),
    out_shape=jax.ShapeDtypeStruct((NPAD, D), jnp.float32),
)

_layer2 = pl.pallas_call(
    _layer2_body,
    grid=(NPAD // BN,),
    in_specs=[_row_spec(D), _row_spec(D), _cnt_spec(),
              _row_spec(D), _full_spec(D, D), _full_spec(D, D), _full_spec(1, D),
              _full_spec(D, D), _full_spec(1, D)],
    out_specs=_row_spec(D),
    out_shape=jax.ShapeDtypeStruct((NPAD, D), jnp.float32),
)


def kernel(x, edge_index, W1_l, b1, W1_r, W2_l, b2, W2_r, W_head, b_head):
    src = edge_index[0]
    dst = edge_index[1]
    pad = EPAD - E
    src_p = jnp.concatenate([src, jnp.zeros((pad,), jnp.int32)])
    dst_p = jnp.concatenate([dst, jnp.full((pad,), N, jnp.int32)])
    src_2d = src_p.reshape(EPAD // CHUNK, CHUNK)
    dst_2d = dst_p.reshape(EPAD // CHUNK, CHUNK)
    x_p = jnp.zeros((NPAD, D), jnp.float32).at[:N].set(x)

    cnts = _degree(dst_p)
    sums1 = _seg_sum(src_2d, dst_2d, x_p)
    h1 = _layer1(sums1[0], sums1[1], cnts, x_p, W1_l, W1_r, b1.reshape(1, D))
    sums2 = _seg_sum(src_2d, dst_2d, h1)
    wh = jnp.zeros((D, D), jnp.float32).at[:, :3].set(W_head)
    bh = jnp.zeros((1, D), jnp.float32).at[0, :3].set(b_head)
    out = _layer2(sums2[0], sums2[1], cnts, h1, W2_l, W2_r, b2.reshape(1, D),
                  wh, bh)
    return out[:N, :3]
